# Initial kernel scaffold; baseline (speedup 1.0000x reference)
import jax
import jax.numpy as jnp
from jax.experimental import pallas as pl


def kernel(actions, table, W1, b1, gamma, beta, W2, b2):
    raise NotImplementedError("write your pallas kernel here")


# speedup vs baseline: 1.0280x; 1.0280x over previous
import functools
import math

import jax
import jax.numpy as jnp
import numpy as np
from jax import lax
from jax.experimental import pallas as pl
from jax.experimental.pallas import tpu as pltpu
from jax.experimental.pallas import tpu_sc as plsc

_NC = 2
_NS = 16
_NW = _NC * _NS

_CH = 128
_NBUF = 4


def _table_mlp_body(table_ref, w1_ref, b1_ref, gamma_ref, beta_ref,
                    w2_ref, b2_ref, out_ref):
    t = table_ref[...]
    h = jnp.dot(t, w1_ref[...], preferred_element_type=jnp.float32)
    h = h + b1_ref[...]
    mean = jnp.mean(h, axis=-1, keepdims=True)
    var = jnp.mean((h - mean) ** 2, axis=-1, keepdims=True)
    h = (h - mean) / jnp.sqrt(var + 1e-5) * gamma_ref[...] + beta_ref[...]
    h = 0.5 * h * (1.0 + lax.erf(h * np.float32(1.0 / math.sqrt(2.0))))
    out_ref[...] = jnp.dot(h, w2_ref[...],
                           preferred_element_type=jnp.float32) + b2_ref[...]


def _table_mlp(table, W1, b1, gamma, beta, W2, b2):
    V, D = table.shape
    return pl.pallas_call(
        _table_mlp_body,
        out_shape=jax.ShapeDtypeStruct((V, D), jnp.float32),
    )(table, W1, b1.reshape(1, D), gamma.reshape(1, D), beta.reshape(1, D),
      W2, b2.reshape(1, D))


def _make_gather(n_rows, V, D):
    n_chunks = n_rows // (_NW * _CH)
    mesh = plsc.VectorSubcoreMesh(core_axis_name="c", subcore_axis_name="s",
                                  num_cores=_NC, num_subcores=_NS)

    @functools.partial(
        pl.kernel,
        out_type=jax.ShapeDtypeStruct((n_rows, D), jnp.float32),
        mesh=mesh,
        scratch_types=[
            pltpu.VMEM((n_chunks, _CH), jnp.int32),
            pltpu.VMEM((_NBUF, _CH, D), jnp.float32),
        ] + [pltpu.SemaphoreType.DMA] * _NBUF,
    )
    def gather(ptable_hbm, idx_hbm, out_hbm, idx_v, rows_v, *gsems):
        wid = lax.axis_index("s") * _NC + lax.axis_index("c")
        base_chunk = wid * n_chunks
        pltpu.sync_copy(idx_hbm.at[pl.ds(base_chunk, n_chunks)], idx_v)

        for b in range(_NBUF):
            pltpu.async_copy(ptable_hbm.at[idx_v.at[b]], rows_v.at[b],
                             gsems[b])

        def group(g, carry):
            for b in range(_NBUF):
                j = g * _NBUF + b
                pltpu.make_async_copy(ptable_hbm.at[idx_v.at[b]],
                                      rows_v.at[b], gsems[b]).wait()
                pltpu.sync_copy(
                    rows_v.at[b],
                    out_hbm.at[pl.ds((base_chunk + j) * _CH, _CH)])

                @pl.when(j + _NBUF < n_chunks)
                def _():
                    pltpu.async_copy(ptable_hbm.at[idx_v.at[j + _NBUF]],
                                     rows_v.at[b], gsems[b])
            return carry

        lax.fori_loop(0, n_chunks // _NBUF, group, 0)

    return gather


def kernel(actions, table, W1, b1, gamma, beta, W2, b2):
    B, S = actions.shape
    V, D = table.shape
    n_rows = B * S

    ptable = _table_mlp(table, W1, b1, gamma, beta, W2, b2)

    idx = actions.reshape(n_rows // _CH, _CH).astype(jnp.int32)
    out_flat = _make_gather(n_rows, V, D)(ptable, idx)
    return out_flat.reshape(B, S, D)


# speedup vs baseline: 6.3998x; 6.2252x over previous
import functools
import math

import jax
import jax.numpy as jnp
import numpy as np
from jax import lax
from jax.experimental import pallas as pl
from jax.experimental.pallas import tpu as pltpu
from jax.experimental.pallas import tpu_sc as plsc

_NC = 2
_NS = 16
_NW = _NC * _NS
_L = 16

_CH = 256
_NBUF = 2
_RUNROLL = 4


def _table_mlp_body(table_ref, w1_ref, b1_ref, gamma_ref, beta_ref,
                    w2_ref, b2_ref, out_ref):
    t = table_ref[...]
    h = jnp.dot(t, w1_ref[...], preferred_element_type=jnp.float32)
    h = h + b1_ref[...]
    mean = jnp.mean(h, axis=-1, keepdims=True)
    var = jnp.mean((h - mean) ** 2, axis=-1, keepdims=True)
    h = (h - mean) / jnp.sqrt(var + 1e-5) * gamma_ref[...] + beta_ref[...]
    h = 0.5 * h * (1.0 + lax.erf(h * np.float32(1.0 / math.sqrt(2.0))))
    out_ref[...] = jnp.dot(h, w2_ref[...],
                           preferred_element_type=jnp.float32) + b2_ref[...]


def _table_mlp(table, W1, b1, gamma, beta, W2, b2):
    V, D = table.shape
    return pl.pallas_call(
        _table_mlp_body,
        out_shape=jax.ShapeDtypeStruct((V, D), jnp.float32),
    )(table, W1, b1.reshape(1, D), gamma.reshape(1, D), beta.reshape(1, D),
      W2, b2.reshape(1, D))


def _make_gather(n_rows, V, D):
    rows_per_w = n_rows // _NW
    n_chunks = rows_per_w // _CH
    mesh = plsc.VectorSubcoreMesh(core_axis_name="c", subcore_axis_name="s",
                                  num_cores=_NC, num_subcores=_NS)

    @functools.partial(
        pl.kernel,
        out_type=jax.ShapeDtypeStruct((n_rows, D), jnp.float32),
        mesh=mesh,
        compiler_params=pltpu.CompilerParams(needs_layout_passes=False),
        scratch_types=[
            pltpu.VMEM((rows_per_w,), jnp.int32),
            pltpu.VMEM((V, D), jnp.float32),
            pltpu.VMEM((_NBUF, _CH, D), jnp.float32),
        ] + [pltpu.SemaphoreType.DMA] * _NBUF,
    )
    def gather(ptable_hbm, idx_hbm, out_hbm, idx_v, pt_v, out_v, *ssems):
        wid = lax.axis_index("s") * _NC + lax.axis_index("c")
        row0 = wid * rows_per_w
        pltpu.sync_copy(ptable_hbm, pt_v)
        pltpu.sync_copy(idx_hbm.at[pl.ds(row0, rows_per_w)], idx_v)
        lanes = lax.iota(jnp.int32, _L)
        cols = [lanes + _L * k for k in range(D // _L)]
        splats = [jnp.full((_L,), rr, jnp.int32) for rr in range(_L)]

        def chunk_group(j, carry):
            for b in range(_NBUF):
                jj = j * _NBUF + b

                @pl.when(jj >= _NBUF)
                def _():
                    pltpu.make_async_copy(
                        out_v.at[b], out_hbm.at[pl.ds(row0, _CH)],
                        ssems[b]).wait()

                def row_group(g, carry2):
                    avec = idx_v[pl.ds(jj * _CH + g * _L, _L)]
                    for rr in range(_L):
                        a = jnp.take_along_axis(
                            avec, splats[rr], axis=0,
                            mode="promise_in_bounds")
                        r = g * _L + rr
                        for k in range(D // _L):
                            out_v[b, r, pl.ds(_L * k, _L)] = (
                                plsc.load_gather(pt_v, [a, cols[k]]))
                    return carry2

                lax.fori_loop(0, _CH // _L, row_group, 0)
                pltpu.async_copy(
                    out_v.at[b],
                    out_hbm.at[pl.ds(row0 + jj * _CH, _CH)], ssems[b])
            return carry

        lax.fori_loop(0, n_chunks // _NBUF, chunk_group, 0)
        for b in range(_NBUF):
            pltpu.make_async_copy(
                out_v.at[b], out_hbm.at[pl.ds(row0, _CH)], ssems[b]).wait()

    return gather


def kernel(actions, table, W1, b1, gamma, beta, W2, b2):
    B, S = actions.shape
    V, D = table.shape
    n_rows = B * S

    ptable = _table_mlp(table, W1, b1, gamma, beta, W2, b2)

    idx = actions.reshape(n_rows).astype(jnp.int32)
    out_flat = _make_gather(n_rows, V, D)(ptable, idx)
    return out_flat.reshape(B, S, D)


# speedup vs baseline: 29.3675x; 4.5888x over previous
import functools
import math

import jax
import jax.numpy as jnp
import numpy as np
from jax import lax
from jax.experimental import pallas as pl
from jax.experimental.pallas import tpu as pltpu
from jax.experimental.pallas import tpu_sc as plsc

_NC = 2
_NS = 16
_NW = _NC * _NS
_L = 16

_CH = 256
_NBUF = 2
_RUNROLL = 4


def _table_mlp_body(table_ref, w1_ref, b1_ref, gamma_ref, beta_ref,
                    w2_ref, b2_ref, out_ref):
    t = table_ref[...]
    h = jnp.dot(t, w1_ref[...], preferred_element_type=jnp.float32)
    h = h + b1_ref[...]
    mean = jnp.mean(h, axis=-1, keepdims=True)
    var = jnp.mean((h - mean) ** 2, axis=-1, keepdims=True)
    h = (h - mean) / jnp.sqrt(var + 1e-5) * gamma_ref[...] + beta_ref[...]
    h = 0.5 * h * (1.0 + lax.erf(h * np.float32(1.0 / math.sqrt(2.0))))
    out_ref[...] = jnp.dot(h, w2_ref[...],
                           preferred_element_type=jnp.float32) + b2_ref[...]


def _table_mlp(table, W1, b1, gamma, beta, W2, b2):
    V, D = table.shape
    return pl.pallas_call(
        _table_mlp_body,
        out_shape=jax.ShapeDtypeStruct((V, D), jnp.float32),
    )(table, W1, b1.reshape(1, D), gamma.reshape(1, D), beta.reshape(1, D),
      W2, b2.reshape(1, D))


def _make_gather(n_rows, V, D):
    rows_per_w = n_rows // _NW
    n_chunks = rows_per_w // _CH
    mesh = plsc.VectorSubcoreMesh(core_axis_name="c", subcore_axis_name="s",
                                  num_cores=_NC, num_subcores=_NS)

    @functools.partial(
        pl.kernel,
        out_type=jax.ShapeDtypeStruct((n_rows, D), jnp.float32),
        mesh=mesh,
        compiler_params=pltpu.CompilerParams(needs_layout_passes=False),
        scratch_types=[
            pltpu.VMEM((rows_per_w,), jnp.int32),
            pltpu.VMEM((V, D), jnp.float32),
            pltpu.VMEM((_NBUF, _CH, D), jnp.float32),
        ] + [pltpu.SemaphoreType.DMA] * _NBUF,
    )
    def gather(ptable_hbm, idx_hbm, out_hbm, idx_v, pt_v, out_v, *ssems):
        wid = lax.axis_index("s") * _NC + lax.axis_index("c")
        row0 = wid * rows_per_w
        pltpu.sync_copy(ptable_hbm, pt_v)
        pltpu.sync_copy(idx_hbm.at[pl.ds(row0, rows_per_w)], idx_v)
        lanes = lax.iota(jnp.int32, _L)
        cols = [lanes + _L * k for k in range(D // _L)]
        splats = [jnp.full((_L,), rr, jnp.int32) for rr in range(_L)]

        def chunk_group(j, carry):
            for b in range(_NBUF):
                jj = j * _NBUF + b

                @pl.when(jj >= _NBUF)
                def _():
                    pltpu.make_async_copy(
                        out_v.at[b], out_hbm.at[pl.ds(row0, _CH)],
                        ssems[b]).wait()

                @functools.partial(plsc.parallel_loop, 0, _CH // _L,
                                   unroll=2)
                def _row_group(g):
                    avec = idx_v[pl.ds(jj * _CH + g * _L, _L)]
                    for rr in range(_L):
                        a = jnp.take_along_axis(
                            avec, splats[rr], axis=0,
                            mode="promise_in_bounds")
                        r = g * _L + rr
                        for k in range(D // _L):
                            out_v[b, r, pl.ds(_L * k, _L)] = (
                                plsc.load_gather(pt_v, [a, cols[k]]))
                pltpu.async_copy(
                    out_v.at[b],
                    out_hbm.at[pl.ds(row0 + jj * _CH, _CH)], ssems[b])
            return carry

        lax.fori_loop(0, n_chunks // _NBUF, chunk_group, 0)
        for b in range(_NBUF):
            pltpu.make_async_copy(
                out_v.at[b], out_hbm.at[pl.ds(row0, _CH)], ssems[b]).wait()

    return gather


def kernel(actions, table, W1, b1, gamma, beta, W2, b2):
    B, S = actions.shape
    V, D = table.shape
    n_rows = B * S

    ptable = _table_mlp(table, W1, b1, gamma, beta, W2, b2)

    idx = actions.reshape(n_rows).astype(jnp.int32)
    out_flat = _make_gather(n_rows, V, D)(ptable, idx)
    return out_flat.reshape(B, S, D)
